# baseline (speedup 1.0000x reference)
import functools

import jax
import jax.numpy as jnp
from jax import lax
from jax.experimental import pallas as pl
from jax.experimental.pallas import tpu as pltpu
from jax.experimental.pallas import tpu_sc as plsc

B = 4096
S = 200
D = 64
C = 16
V = 1000000
NC = 2
NS = 16
NW = NC * NS
BPW = B // NW
CH0, CH1 = 104, 96
LANES = 16
PACK = 8
VP = 131072
TBLK = 2048
NBLK = VP // TBLK
LAST = (V - 1) // TBLK


def _mm_body(*refs):
    et_refs, w8_ref, out_ref = refs[:PACK], refs[PACK], refs[PACK + 1]
    es = jnp.concatenate([et_refs[a][...] for a in range(PACK)], axis=0)
    out_ref[...] = lax.dot_general(
        es, w8_ref[...], (((0,), (0,)), ((), ())),
        preferred_element_type=jnp.float32)


def _mk_in_spec(a):
    return pl.BlockSpec((D, TBLK),
                        lambda v: (0, jnp.minimum(a * NBLK + v, LAST)))


_pack_table = pl.pallas_call(
    _mm_body,
    grid=(NBLK,),
    in_specs=[_mk_in_spec(a) for a in range(PACK)]
    + [pl.BlockSpec((PACK * D, PACK * C), lambda v: (0, 0))],
    out_specs=pl.BlockSpec((TBLK, PACK * C), lambda v: (v, 0)),
    out_shape=jax.ShapeDtypeStruct((VP, PACK * C), jnp.float32),
)


NSLOT = 4


def _sc_body(*refs):
    (ids_hbm, tp_hbm, bias_hbm, out_hbm) = refs[:4]
    idx_v = refs[4]
    bufsA = refs[5:5 + NSLOT]
    bufsB = refs[5 + NSLOT:5 + 2 * NSLOT]
    bias_v, out_v = refs[5 + 2 * NSLOT:7 + 2 * NSLOT]
    semsA = refs[7 + 2 * NSLOT:7 + 3 * NSLOT]
    semsB = refs[7 + 3 * NSLOT:7 + 4 * NSLOT]

    wid = lax.axis_index("s") * NC + lax.axis_index("c")
    base = wid * BPW
    pltpu.sync_copy(ids_hbm.at[pl.ds(base * S, BPW * S)], idx_v)
    pltpu.sync_copy(bias_hbm, bias_v)

    def mkidx(k, carry):
        for j in range(4):
            o = (k * 4 + j) * LANES
            v = idx_v[pl.ds(o, LANES)]
            idx_v[pl.ds(o, LANES)] = ((v & (VP - 1)) << 3) | (v >> 17)
        return carry

    lax.fori_loop(0, BPW * S // (4 * LANES), mkidx, 0)

    def fire(r, slot):
        off = pl.multiple_of(r * S, 8)
        pltpu.async_copy(tp_hbm.at[idx_v.at[pl.ds(off, CH0)]],
                         bufsA[slot], semsA[slot])
        off1 = pl.multiple_of(r * S + CH0, 8)
        pltpu.async_copy(tp_hbm.at[idx_v.at[pl.ds(off1, CH1)]],
                         bufsB[slot], semsB[slot])

    for r0 in range(NSLOT):
        fire(r0, r0)

    bias_vec = bias_v[...]
    inv_s = jnp.float32(1.0 / S)
    zero = jnp.zeros((LANES,), jnp.float32)

    def step(i, slot):
        r = i * NSLOT + slot
        pltpu.make_async_copy(tp_hbm.at[idx_v.at[pl.ds(0, CH0)]],
                              bufsA[slot], semsA[slot]).wait()
        pltpu.make_async_copy(tp_hbm.at[idx_v.at[pl.ds(0, CH1)]],
                              bufsB[slot], semsB[slot]).wait()

        accs = [zero, zero, zero, zero]
        for t in range(S):
            buf = bufsA[slot] if t < CH0 else bufsB[slot]
            row = t if t < CH0 else t - CH0
            accs[t % 4] = accs[t % 4] + buf[row, :]
        acc = (accs[0] + accs[1]) + (accs[2] + accs[3])

        @pl.when(r + NSLOT < BPW)
        def _():
            fire(r + NSLOT, slot)

        out_v[r, :] = bias_vec + acc * inv_s

    def outer(i, carry):
        for slot in range(NSLOT):
            step(i, slot)
        return carry

    lax.fori_loop(0, BPW // NSLOT, outer, 0)
    pltpu.sync_copy(out_v, out_hbm.at[pl.ds(base, BPW)])


@functools.partial(
    pl.kernel,
    out_type=jax.ShapeDtypeStruct((B, C), jnp.float32),
    mesh=plsc.VectorSubcoreMesh(core_axis_name="c", subcore_axis_name="s",
                                num_cores=NC, num_subcores=NS),
    compiler_params=pltpu.CompilerParams(use_tc_tiling_on_sc=False),
    scratch_types=[pltpu.VMEM((BPW * S,), jnp.int32)]
    + [pltpu.VMEM((CH0, C), jnp.float32)] * NSLOT
    + [pltpu.VMEM((CH1, C), jnp.float32)] * NSLOT
    + [pltpu.VMEM((C,), jnp.float32), pltpu.VMEM((BPW, C), jnp.float32)]
    + [pltpu.SemaphoreType.DMA] * (2 * NSLOT),
)
def _sc_pool(*refs):
    _sc_body(*refs)


def kernel(input_ids, embedding, W, b):
    ids_flat = input_ids.reshape(-1).astype(jnp.int32)
    wt = jnp.asarray(W, jnp.float32).T
    w8 = jnp.kron(jnp.eye(PACK, dtype=jnp.float32), wt)
    embt = embedding.T
    tpack = _pack_table(*([embt] * PACK), w8)
    tview = tpack.reshape(VP * PACK, C)
    return _sc_pool(ids_flat, tview, b)


# speedup vs baseline: 1.0843x; 1.0843x over previous
import functools

import jax
import jax.numpy as jnp
from jax import lax
from jax.experimental import pallas as pl
from jax.experimental.pallas import tpu as pltpu
from jax.experimental.pallas import tpu_sc as plsc

B = 4096
S = 200
D = 64
C = 16
V = 1000000
NC = 2
NS = 16
NW = NC * NS
BPW = B // NW
CH0, CH1 = 104, 96
LANES = 16
PACK = 8
VP = 131072
TBLK = 4096
NBLK = VP // TBLK
LAST = (V - 1) // TBLK


def _mm_body(*refs):
    et_refs, w8_ref, out_ref = refs[:PACK], refs[PACK], refs[PACK + 1]
    es = jnp.concatenate([et_refs[a][...] for a in range(PACK)], axis=0)
    out_ref[...] = lax.dot_general(
        es, w8_ref[...], (((0,), (0,)), ((), ())),
        preferred_element_type=jnp.float32)


def _mk_in_spec(a):
    return pl.BlockSpec((D, TBLK),
                        lambda v: (0, jnp.minimum(a * NBLK + v, LAST)))


_pack_table = pl.pallas_call(
    _mm_body,
    grid=(NBLK,),
    in_specs=[_mk_in_spec(a) for a in range(PACK)]
    + [pl.BlockSpec((PACK * D, PACK * C), lambda v: (0, 0))],
    out_specs=pl.BlockSpec((TBLK, PACK * C), lambda v: (v, 0)),
    out_shape=jax.ShapeDtypeStruct((VP, PACK * C), jnp.float32),
)


NSLOT = 4


def _sc_body(*refs):
    (ids_hbm, tp_hbm, bias_hbm, out_hbm) = refs[:4]
    idx_v = refs[4]
    bufs = refs[5:5 + NSLOT]
    bias_v, out_v = refs[5 + NSLOT:7 + NSLOT]
    sems = refs[7 + NSLOT:7 + 2 * NSLOT]

    wid = lax.axis_index("s") * NC + lax.axis_index("c")
    base = wid * BPW
    pltpu.sync_copy(ids_hbm.at[pl.ds(base * S, BPW * S)], idx_v)
    pltpu.sync_copy(bias_hbm, bias_v)

    def mkidx(k, carry):
        for j in range(4):
            o = (k * 4 + j) * LANES
            v = idx_v[pl.ds(o, LANES)]
            idx_v[pl.ds(o, LANES)] = ((v & (VP - 1)) << 3) | (v >> 17)
        return carry

    lax.fori_loop(0, BPW * S // (4 * LANES), mkidx, 0)

    def fire(r, slot):
        off = pl.multiple_of(r * S, 8)
        pltpu.async_copy(tp_hbm.at[idx_v.at[pl.ds(off, CH0)]],
                         bufs[slot].at[pl.ds(0, CH0)], sems[slot])
        off1 = pl.multiple_of(r * S + CH0, 8)
        pltpu.async_copy(tp_hbm.at[idx_v.at[pl.ds(off1, CH1)]],
                         bufs[slot].at[pl.ds(CH0, CH1)], sems[slot])

    for r0 in range(NSLOT):
        fire(r0, r0)

    bias_vec = bias_v[...]
    inv_s = jnp.float32(1.0 / S)
    zero = jnp.zeros((LANES,), jnp.float32)

    def step(i, slot):
        r = i * NSLOT + slot
        pltpu.make_async_copy(tp_hbm.at[idx_v.at[pl.ds(0, S)]],
                              bufs[slot], sems[slot]).wait()

        accs = [zero, zero, zero, zero]
        for t in range(S):
            accs[t % 4] = accs[t % 4] + bufs[slot][t, :]
        acc = (accs[0] + accs[1]) + (accs[2] + accs[3])

        @pl.when(r + NSLOT < BPW)
        def _():
            fire(r + NSLOT, slot)

        out_v[r, :] = bias_vec + acc * inv_s

    def outer(i, carry):
        for slot in range(NSLOT):
            step(i, slot)
        return carry

    lax.fori_loop(0, BPW // NSLOT, outer, 0)
    pltpu.sync_copy(out_v, out_hbm.at[pl.ds(base, BPW)])


@functools.partial(
    pl.kernel,
    out_type=jax.ShapeDtypeStruct((B, C), jnp.float32),
    mesh=plsc.VectorSubcoreMesh(core_axis_name="c", subcore_axis_name="s",
                                num_cores=NC, num_subcores=NS),
    compiler_params=pltpu.CompilerParams(use_tc_tiling_on_sc=False),
    scratch_types=[pltpu.VMEM((BPW * S,), jnp.int32)]
    + [pltpu.VMEM((S, C), jnp.float32)] * NSLOT
    + [pltpu.VMEM((C,), jnp.float32), pltpu.VMEM((BPW, C), jnp.float32)]
    + [pltpu.SemaphoreType.DMA] * NSLOT,
)
def _sc_pool(*refs):
    _sc_body(*refs)


def kernel(input_ids, embedding, W, b):
    ids_flat = input_ids.reshape(-1).astype(jnp.int32)
    wt = jnp.asarray(W, jnp.float32).T
    w8 = jnp.kron(jnp.eye(PACK, dtype=jnp.float32), wt)
    embt = embedding.T
    tpack = _pack_table(*([embt] * PACK), w8)
    tview = tpack.reshape(VP * PACK, C)
    return _sc_pool(ids_flat, tview, b)


# speedup vs baseline: 1.1035x; 1.0177x over previous
import functools

import jax
import jax.numpy as jnp
from jax import lax
from jax.experimental import pallas as pl
from jax.experimental.pallas import tpu as pltpu
from jax.experimental.pallas import tpu_sc as plsc

B = 4096
S = 200
D = 64
C = 16
V = 1000000
NC = 2
NS = 16
NW = NC * NS
BPW = B // NW
CH0, CH1 = 104, 96
LANES = 16
PACK = 8
VP = 131072
TBLK = 4096
NBLK = VP // TBLK
LAST = (V - 1) // TBLK


def _mm_body(*refs):
    et_refs, w8_ref, out_ref = refs[:PACK], refs[PACK], refs[PACK + 1]
    es = jnp.concatenate([et_refs[a][...] for a in range(PACK)], axis=0)
    out_ref[...] = lax.dot_general(
        es, w8_ref[...], (((0,), (0,)), ((), ())),
        preferred_element_type=jnp.float32)


def _mk_in_spec(a):
    return pl.BlockSpec((D, TBLK),
                        lambda v: (0, jnp.minimum(a * NBLK + v, LAST)))


_pack_table = pl.pallas_call(
    _mm_body,
    grid=(NBLK,),
    in_specs=[_mk_in_spec(a) for a in range(PACK)]
    + [pl.BlockSpec((PACK * D, PACK * C), lambda v: (0, 0))],
    out_specs=pl.BlockSpec((TBLK, PACK * C), lambda v: (v, 0)),
    out_shape=jax.ShapeDtypeStruct((VP, PACK * C), jnp.float32),
)


NSLOT = 4


def _sc_body(*refs):
    (ids_hbm, tp_hbm, bias_hbm, out_hbm) = refs[:4]
    idx_v = refs[4]
    bufsA = refs[5:5 + NSLOT]
    bufsB = refs[5 + NSLOT:5 + 2 * NSLOT]
    bias_v, out_v = refs[5 + 2 * NSLOT:7 + 2 * NSLOT]
    semsA = refs[7 + 2 * NSLOT:7 + 3 * NSLOT]
    semsB = refs[7 + 3 * NSLOT:7 + 4 * NSLOT]

    wid = lax.axis_index("s") * NC + lax.axis_index("c")
    base = wid * BPW
    pltpu.sync_copy(ids_hbm.at[pl.ds(base * S, BPW * S)], idx_v)
    pltpu.sync_copy(bias_hbm, bias_v)

    def mkidx(k, carry):
        for j in range(4):
            o = (k * 4 + j) * LANES
            v = idx_v[pl.ds(o, LANES)]
            idx_v[pl.ds(o, LANES)] = ((v & (VP - 1)) << 3) | (v >> 17)
        return carry

    lax.fori_loop(0, BPW * S // (4 * LANES), mkidx, 0)

    def fire(r, slot):
        off = pl.multiple_of(r * S, 8)
        pltpu.async_copy(tp_hbm.at[idx_v.at[pl.ds(off, CH0)]],
                         bufsA[slot], semsA[slot])
        off1 = pl.multiple_of(r * S + CH0, 8)
        pltpu.async_copy(tp_hbm.at[idx_v.at[pl.ds(off1, CH1)]],
                         bufsB[slot], semsB[slot])

    for r0 in range(NSLOT):
        fire(r0, r0)

    bias_vec = bias_v[...]
    inv_s = jnp.float32(1.0 / S)
    zero = jnp.zeros((LANES,), jnp.float32)

    def step(i, slot):
        r = i * NSLOT + slot
        pltpu.make_async_copy(tp_hbm.at[idx_v.at[pl.ds(0, CH0)]],
                              bufsA[slot], semsA[slot]).wait()
        pltpu.make_async_copy(tp_hbm.at[idx_v.at[pl.ds(0, CH1)]],
                              bufsB[slot], semsB[slot]).wait()

        accs = [zero, zero, zero, zero]
        for t in range(S):
            buf = bufsA[slot] if t < CH0 else bufsB[slot]
            row = t if t < CH0 else t - CH0
            accs[t % 4] = accs[t % 4] + buf[row, :]
        acc = (accs[0] + accs[1]) + (accs[2] + accs[3])

        @pl.when(r + NSLOT < BPW)
        def _():
            fire(r + NSLOT, slot)

        out_v[r, :] = bias_vec + acc * inv_s

    def outer(i, carry):
        for slot in range(NSLOT):
            step(i, slot)
        return carry

    lax.fori_loop(0, BPW // NSLOT, outer, 0)
    pltpu.sync_copy(out_v, out_hbm.at[pl.ds(base, BPW)])


@functools.partial(
    pl.kernel,
    out_type=jax.ShapeDtypeStruct((B, C), jnp.float32),
    mesh=plsc.VectorSubcoreMesh(core_axis_name="c", subcore_axis_name="s",
                                num_cores=NC, num_subcores=NS),
    compiler_params=pltpu.CompilerParams(use_tc_tiling_on_sc=False),
    scratch_types=[pltpu.VMEM((BPW * S,), jnp.int32)]
    + [pltpu.VMEM((CH0, C), jnp.float32)] * NSLOT
    + [pltpu.VMEM((CH1, C), jnp.float32)] * NSLOT
    + [pltpu.VMEM((C,), jnp.float32), pltpu.VMEM((BPW, C), jnp.float32)]
    + [pltpu.SemaphoreType.DMA] * (2 * NSLOT),
)
def _sc_pool(*refs):
    _sc_body(*refs)


def kernel(input_ids, embedding, W, b):
    ids_flat = input_ids.reshape(-1).astype(jnp.int32)
    wt = jnp.asarray(W, jnp.float32).T
    w8 = jnp.kron(jnp.eye(PACK, dtype=jnp.float32), wt)
    embt = embedding.T
    tpack = _pack_table(*([embt] * PACK), w8)
    tview = tpack.reshape(VP * PACK, C)
    return _sc_pool(ids_flat, tview, b)
